# Initial kernel scaffold; baseline (speedup 1.0000x reference)
import jax
import jax.numpy as jnp
from jax.experimental import pallas as pl


def kernel(indices, table):
    raise NotImplementedError("write your pallas kernel here")


# speedup vs baseline: 4.5333x; 4.5333x over previous
import functools

import jax
import jax.numpy as jnp
from jax import lax
from jax.experimental import pallas as pl
from jax.experimental.pallas import tpu as pltpu
from jax.experimental.pallas import tpu_sc as plsc

_NUM_CORES = 2
_NUM_SUBCORES = 16
_NW = _NUM_CORES * _NUM_SUBCORES

_D = 64
_B = 4096 * 50
_B_PER_W = _B // _NW
_CHUNK = 128
_N_CHUNKS = _B_PER_W // _CHUNK
_NBUF = 2


@functools.partial(
    pl.kernel,
    out_type=jax.ShapeDtypeStruct((_B, _D), jnp.float32),
    mesh=plsc.VectorSubcoreMesh(core_axis_name="c", subcore_axis_name="s"),
    compiler_params=pltpu.CompilerParams(use_tc_tiling_on_sc=False),
    scratch_types=[
        pltpu.VMEM((_N_CHUNKS, _CHUNK), jnp.int32),
        pltpu.VMEM((_NBUF, _CHUNK, _D), jnp.float32),
        pltpu.SemaphoreType.DMA((_NBUF,)),
    ],
)
def _emb_lookup(table_hbm, idx_hbm, out_hbm, idx_v, rows_v, gsems):
    wid = lax.axis_index("s") * _NUM_CORES + lax.axis_index("c")
    base = wid * _B_PER_W

    pltpu.sync_copy(idx_hbm.at[wid], idx_v)

    for b in range(_NBUF):
        pltpu.async_copy(table_hbm.at[idx_v.at[b]], rows_v.at[b], gsems.at[b])

    @pl.loop(0, _N_CHUNKS, step=_NBUF)
    def _steps(j0):
        for b in range(_NBUF):
            j = j0 + b
            pltpu.make_async_copy(
                table_hbm.at[idx_v.at[j]], rows_v.at[b], gsems.at[b]
            ).wait()
            pltpu.sync_copy(
                rows_v.at[b], out_hbm.at[pl.ds(base + j * _CHUNK, _CHUNK)]
            )
            jn = j + _NBUF

            @pl.when(jn < _N_CHUNKS)
            def _():
                pltpu.async_copy(
                    table_hbm.at[idx_v.at[jn]], rows_v.at[b], gsems.at[b]
                )


def kernel(indices, table):
    idx3 = indices.reshape(_NW, _N_CHUNKS, _CHUNK).astype(jnp.int32)
    out = _emb_lookup(table, idx3)
    return out.reshape(indices.shape + (_D,))


# speedup vs baseline: 4.6753x; 1.0313x over previous
import functools

import jax
import jax.numpy as jnp
from jax import lax
from jax.experimental import pallas as pl
from jax.experimental.pallas import tpu as pltpu
from jax.experimental.pallas import tpu_sc as plsc

_NUM_CORES = 2
_NUM_SUBCORES = 16
_NW = _NUM_CORES * _NUM_SUBCORES

_D = 64
_B = 4096 * 50
_B_PER_W = _B // _NW
_CHUNK = 128
_N_CHUNKS = _B_PER_W // _CHUNK
_NBUF = 10
_LOOKAHEAD = _NBUF // 2


@functools.partial(
    pl.kernel,
    out_type=jax.ShapeDtypeStruct((_B, _D), jnp.float32),
    mesh=plsc.VectorSubcoreMesh(core_axis_name="c", subcore_axis_name="s"),
    compiler_params=pltpu.CompilerParams(use_tc_tiling_on_sc=False),
    scratch_types=[
        pltpu.VMEM((_N_CHUNKS, _CHUNK), jnp.int32),
        pltpu.VMEM((_NBUF, _CHUNK, _D), jnp.float32),
        pltpu.SemaphoreType.DMA((_NBUF,)),
        pltpu.SemaphoreType.DMA((_NBUF,)),
    ],
)
def _emb_lookup(table_hbm, idx_hbm, out_hbm, idx_v, rows_v, gsems, wsems):
    wid = lax.axis_index("s") * _NUM_CORES + lax.axis_index("c")
    base = wid * _B_PER_W

    pltpu.sync_copy(idx_hbm.at[wid], idx_v)

    for b in range(_LOOKAHEAD):
        pltpu.async_copy(table_hbm.at[idx_v.at[b]], rows_v.at[b], gsems.at[b])

    @pl.loop(0, _N_CHUNKS, step=_NBUF)
    def _steps(j0):
        for b in range(_NBUF):
            j = j0 + b
            pltpu.make_async_copy(
                table_hbm.at[idx_v.at[j]], rows_v.at[b], gsems.at[b]
            ).wait()
            pltpu.async_copy(
                rows_v.at[b], out_hbm.at[pl.ds(base + j * _CHUNK, _CHUNK)],
                wsems.at[b],
            )
            jn = j + _LOOKAHEAD
            bn = (b + _LOOKAHEAD) % _NBUF

            @pl.when(jn < _N_CHUNKS)
            def _():
                jprev = jn - _NBUF

                @pl.when(jprev >= 0)
                def _():
                    pltpu.make_async_copy(
                        rows_v.at[bn],
                        out_hbm.at[pl.ds(base + jprev * _CHUNK, _CHUNK)],
                        wsems.at[bn],
                    ).wait()

                pltpu.async_copy(
                    table_hbm.at[idx_v.at[jn]], rows_v.at[bn], gsems.at[bn]
                )

    for b in range(_NBUF):
        j = _N_CHUNKS - _NBUF + b
        pltpu.make_async_copy(
            rows_v.at[b], out_hbm.at[pl.ds(base + j * _CHUNK, _CHUNK)],
            wsems.at[b],
        ).wait()


def kernel(indices, table):
    idx3 = indices.reshape(_NW, _N_CHUNKS, _CHUNK).astype(jnp.int32)
    out = _emb_lookup(table, idx3)
    return out.reshape(indices.shape + (_D,))
